# baseline (speedup 1.0000x reference)
import functools
import jax
import jax.numpy as jnp
from jax.experimental import pallas as pl
from jax.experimental.pallas import tpu as pltpu

_CUTOFF = 0.15
_BI = 512
_BJ = 512


def _nbr_kernel(p_ref, pt_ref, out_ref, *, bi, bj):
    a8 = p_ref[...]
    bt8 = pt_ref[...]
    sqi = a8[:, 3:4]
    sqj = bt8[3:4, :]
    a = a8.astype(jnp.bfloat16)
    s = jnp.where(
        jax.lax.broadcasted_iota(jnp.int32, (8, 1), 0) < 3, -2.0, 0.0)
    b = (bt8 * s).astype(jnp.bfloat16)
    dot = jax.lax.dot_general(
        a, b, (((1,), (0,)), ((), ())),
        preferred_element_type=jnp.float32)
    d2 = (sqi + sqj) + dot
    keep_in = (d2 < _CUTOFF * _CUTOFF) & (d2 > 0.0)
    val = jnp.where(keep_in, d2 * jax.lax.rsqrt(d2), 0.0)
    i = pl.program_id(0)
    j = pl.program_id(1)

    @pl.when(i != j)
    def _():
        out_ref[...] = val

    @pl.when(i == j)
    def _():
        r = jax.lax.broadcasted_iota(jnp.int32, (bi, bj), 0)
        c = jax.lax.broadcasted_iota(jnp.int32, (bi, bj), 1)
        out_ref[...] = jnp.where(r == c, 0.0, val)


def kernel(positions):
    n = positions.shape[0]
    sq = jnp.sum(positions * positions, axis=1, keepdims=True)
    p = jnp.concatenate([positions, sq, jnp.zeros((n, 4), jnp.float32)], 1)
    pt = p.T
    return pl.pallas_call(
        functools.partial(_nbr_kernel, bi=_BI, bj=_BJ),
        grid=(n // _BI, n // _BJ),
        in_specs=[
            pl.BlockSpec((_BI, 8), lambda i, j: (i, 0)),
            pl.BlockSpec((8, _BJ), lambda i, j: (0, j)),
        ],
        out_specs=pl.BlockSpec((_BI, _BJ), lambda i, j: (i, j)),
        out_shape=jax.ShapeDtypeStruct((n, n), jnp.float32),
        compiler_params=pltpu.CompilerParams(
            dimension_semantics=("parallel", "parallel")),
    )(p, pt)


# speedup vs baseline: 1.9299x; 1.9299x over previous
import functools
import jax
import jax.numpy as jnp
from jax.experimental import pallas as pl
from jax.experimental.pallas import tpu as pltpu

_CUTOFF = 0.15
_BLK = 512


def _nbr_kernel(prow_ref, pfull_ref, out_ref, bt_ref, sqr_ref, *, blk):
    @pl.when(pl.program_id(0) == 0)
    def _prep():
        pt = pfull_ref[...].T
        bt_ref[...] = (pt * (-2.0)).astype(jnp.bfloat16)
        x, y, z = pt[0:1, :], pt[1:2, :], pt[2:3, :]
        sqr_ref[...] = x * x + y * y + z * z

    arow = prow_ref[...]
    xi, yi, zi = arow[:, 0:1], arow[:, 1:2], arow[:, 2:3]
    sqi = xi * xi + yi * yi + zi * zi
    sqj = sqr_ref[...]
    a = arow.astype(jnp.bfloat16)
    dot = jax.lax.dot_general(
        a, bt_ref[...], (((1,), (0,)), ((), ())),
        preferred_element_type=jnp.float32)
    d2 = (sqi + sqj) + dot
    keep_in = (d2 < _CUTOFF * _CUTOFF) & (d2 > 0.0)
    out_ref[...] = jnp.where(keep_in, d2 * jax.lax.rsqrt(d2), 0.0)
    j0 = pl.program_id(0) * blk
    r = jax.lax.broadcasted_iota(jnp.int32, (blk, blk), 0)
    c = jax.lax.broadcasted_iota(jnp.int32, (blk, blk), 1)
    keep = jnp.where(r == c, 0.0, 1.0)
    out_ref[:, pl.ds(j0, blk)] = out_ref[:, pl.ds(j0, blk)] * keep


def kernel(positions):
    n = positions.shape[0]
    return pl.pallas_call(
        functools.partial(_nbr_kernel, blk=_BLK),
        grid=(n // _BLK,),
        in_specs=[
            pl.BlockSpec((_BLK, 3), lambda i: (i, 0)),
            pl.BlockSpec((n, 3), lambda i: (0, 0)),
        ],
        out_specs=pl.BlockSpec((_BLK, n), lambda i: (i, 0)),
        out_shape=jax.ShapeDtypeStruct((n, n), jnp.float32),
        scratch_shapes=[
            pltpu.VMEM((3, n), jnp.bfloat16),
            pltpu.VMEM((1, n), jnp.float32),
        ],
        compiler_params=pltpu.CompilerParams(
            dimension_semantics=("arbitrary",)),
    )(positions, positions)
